# baseline (speedup 1.0000x reference)
import functools

import jax
import jax.numpy as jnp
from jax import lax
from jax.experimental import pallas as pl
from jax.experimental.pallas import tpu as pltpu
from jax.experimental.pallas import tpu_sc as plsc


@functools.lru_cache(maxsize=None)
def _make_sc_agg(n, d, e, with_cnt):
    info = plsc.get_sparse_core_info()
    nc, ns = info.num_cores, info.num_subcores
    nw = nc * ns
    epw = e // nw
    K = 80
    nchunks = epw // K
    npairs = (nchunks + 1) // 2
    n_pad = ((n + K * ns - 1) // (K * ns)) * (K * ns)
    rows_per_tile = n_pad // ns
    nwb = rows_per_tile // K

    mesh = plsc.VectorSubcoreMesh(core_axis_name="c", subcore_axis_name="s")
    out_type = [jax.ShapeDtypeStruct((nc, n_pad, d), jnp.float32)]
    if with_cnt:
        out_type.append(jax.ShapeDtypeStruct((nw * n,), jnp.float32))

    scratch = [
        pltpu.VMEM((nchunks, K), jnp.int32),
        pltpu.VMEM((K,), jnp.int32),
        pltpu.VMEM((K,), jnp.int32),
        pltpu.VMEM((K,), jnp.int32),
        pltpu.VMEM((K,), jnp.int32),
        pltpu.VMEM((K, d), jnp.float32),
        pltpu.VMEM((K, d), jnp.float32),
        pltpu.VMEM_SHARED((n_pad, d), jnp.float32),
        pltpu.SemaphoreType.DMA,
        pltpu.SemaphoreType.DMA,
        pltpu.SemaphoreType.DMA,
        pltpu.SemaphoreType.DMA,
        pltpu.SemaphoreType.DMA,
        pltpu.SemaphoreType.DMA,
    ]
    if with_cnt:
        scratch.append(pltpu.VMEM((n,), jnp.float32))

    def body(x_hbm, src1_hbm, eidx_hbm, agg_hbm, *rest):
        if with_cnt:
            (cnt_hbm, dst2d, sib0, sib1, sib2, sib3, rows_a, rows_b, agg_sh,
             isem0, isem1, isem2, isem3, gsem_a, gsem_b, cnt_v) = rest
        else:
            (dst2d, sib0, sib1, sib2, sib3, rows_a, rows_b, agg_sh,
             isem0, isem1, isem2, isem3, gsem_a, gsem_b) = rest
        c = lax.axis_index("c")
        s = lax.axis_index("s")
        wid = s * nc + c
        ebase = wid * epw
        z16 = jnp.zeros((16,), jnp.float32)

        pltpu.sync_copy(eidx_hbm.at[wid], dst2d)

        def zrow(r, carry):
            for c8 in range(d // 16):
                rows_a[r, pl.ds(c8 * 16, 16)] = z16
            return carry
        lax.fori_loop(0, K, zrow, 0)
        row0 = s * rows_per_tile
        for j in range(nwb):
            pltpu.sync_copy(rows_a, agg_sh.at[pl.ds(row0 + j * K, K)])
        if with_cnt:
            def zcnt(i, carry):
                cnt_v[pl.ds(i * 16, 16)] = z16
                return carry
            lax.fori_loop(0, n // 16, zcnt, 0)
        plsc.subcore_barrier()

        sibs = ((sib0, isem0), (sib1, isem1), (sib2, isem2), (sib3, isem3))
        for j in range(4):
            pltpu.async_copy(src1_hbm.at[pl.ds(ebase + j * K, K)],
                             sibs[j][0], sibs[j][1])
        pltpu.make_async_copy(src1_hbm.at[pl.ds(ebase, K)],
                              sib0, isem0).wait()
        pltpu.async_copy(x_hbm.at[sib0], rows_a, gsem_a)
        ones = jnp.ones((16,), jnp.float32)
        rbufs = ((rows_a, gsem_a), (rows_b, gsem_b))

        def quad(g, carry):
            for b in range(4):
                i = 4 * g + b
                sib, isem = sibs[b]
                nsib, nisem = sibs[(b + 1) % 4]
                rows, gsem = rbufs[b % 2]
                nrows, ngsem = rbufs[(b + 1) % 2]

                @pl.when(i + 1 < nchunks)
                def _():
                    pltpu.make_async_copy(
                        src1_hbm.at[pl.ds(ebase + (i + 1) * K, K)],
                        nsib, nisem).wait()
                    pltpu.async_copy(x_hbm.at[nsib], nrows, ngsem)

                @pl.when(i < nchunks)
                def _():
                    pltpu.make_async_copy(x_hbm.at[sib], rows, gsem).wait()

                    @pl.when(i + 4 < nchunks)
                    def _():
                        pltpu.async_copy(
                            src1_hbm.at[pl.ds(ebase + (i + 4) * K, K)],
                            sib, isem)
                    pltpu.sync_copy(rows, agg_sh.at[dst2d.at[i]], add=True)
                    if with_cnt:
                        for j in range(K // 16):
                            idx = dst2d[i, pl.ds(j * 16, 16)]
                            plsc.addupdate_scatter(cnt_v, [idx], ones)
            return carry
        lax.fori_loop(0, (nchunks + 3) // 4, quad, 0)
        plsc.subcore_barrier()

        for j in range(nwb):
            r = row0 + j * K
            pltpu.sync_copy(agg_sh.at[pl.ds(r, K)], rows_a)
            pltpu.sync_copy(rows_a, agg_hbm.at[c, pl.ds(r, K)])
        if with_cnt:
            pltpu.sync_copy(cnt_v, cnt_hbm.at[pl.ds(wid * n, n)])

    ot = tuple(out_type) if with_cnt else out_type[0]
    return pl.kernel(body, out_type=ot, mesh=mesh, scratch_types=scratch,
                     compiler_params=pltpu.CompilerParams(
                         needs_layout_passes=False))


@functools.lru_cache(maxsize=None)
def _make_dense(n, d, h_dim, nc, nw, relu):
    R = 1000
    grid = (n // R,)

    def body(agg_ref, cntp_ref, x_ref, wl_ref, b_ref, wr_ref, out_ref):
        cnt = jnp.sum(cntp_ref[...], axis=1)
        inv = 1.0 / jnp.maximum(cnt, 1.0)
        agg = (agg_ref[0] + agg_ref[1]) * inv[:, None]
        y = lax.dot_general(agg, wl_ref[...], (((1,), (1,)), ((), ())),
                            preferred_element_type=jnp.float32)
        y = y + b_ref[...]
        y = y + lax.dot_general(x_ref[...], wr_ref[...],
                                (((1,), (1,)), ((), ())),
                                preferred_element_type=jnp.float32)
        out_ref[...] = jnp.maximum(y, 0.0) if relu else y

    return pl.pallas_call(
        body,
        grid=grid,
        in_specs=[
            pl.BlockSpec((nc, R, d), lambda i: (0, i, 0)),
            pl.BlockSpec((R, nw), lambda i: (i, 0)),
            pl.BlockSpec((R, d), lambda i: (i, 0)),
            pl.BlockSpec((h_dim, d), lambda i: (0, 0)),
            pl.BlockSpec((1, h_dim), lambda i: (0, 0)),
            pl.BlockSpec((h_dim, d), lambda i: (0, 0)),
        ],
        out_specs=pl.BlockSpec((R, h_dim), lambda i: (i, 0)),
        out_shape=jax.ShapeDtypeStruct((n, h_dim), jnp.float32),
    )


def kernel(x, edge_index, W1_l, b1, W1_r, W2_l, b2, W2_r):
    n, d = x.shape
    e = edge_index.shape[1]
    h_dim = W1_l.shape[0]
    o_dim = W2_l.shape[0]
    info = plsc.get_sparse_core_info()
    nc, nw = info.num_cores, info.num_cores * info.num_subcores

    epw = e // nw
    K = 80
    nchunks = epw // K
    src_nodes = edge_index[0]
    dst2d = edge_index[1].reshape(nw, nchunks, K)

    aggp1, cntp = _make_sc_agg(n, d, e, True)(x, src_nodes, dst2d)
    cntp_t = cntp.reshape(nw, n).T
    h = _make_dense(n, d, h_dim, nc, nw, True)(
        aggp1, cntp_t, x, W1_l, b1.reshape(1, -1), W1_r)
    aggp2 = _make_sc_agg(n, h_dim, e, False)(h, src_nodes, dst2d)
    out = _make_dense(n, h_dim, o_dim, nc, nw, False)(
        aggp2, cntp_t, h, W2_l, b2.reshape(1, -1), W2_r)
    return (out, edge_index)


# speedup vs baseline: 1.0038x; 1.0038x over previous
import functools

import jax
import jax.numpy as jnp
from jax import lax
from jax.experimental import pallas as pl
from jax.experimental.pallas import tpu as pltpu
from jax.experimental.pallas import tpu_sc as plsc


@functools.lru_cache(maxsize=None)
def _make_sc_agg(n, d, e, with_cnt):
    info = plsc.get_sparse_core_info()
    nc, ns = info.num_cores, info.num_subcores
    nw = nc * ns
    epw = e // nw
    K = 80
    nchunks = epw // K
    npairs = (nchunks + 1) // 2
    n_pad = ((n + K * ns - 1) // (K * ns)) * (K * ns)
    rows_per_tile = n_pad // ns
    nwb = rows_per_tile // K

    mesh = plsc.VectorSubcoreMesh(core_axis_name="c", subcore_axis_name="s")
    out_type = [jax.ShapeDtypeStruct((nc, n_pad, d), jnp.float32)]
    if with_cnt:
        out_type.append(jax.ShapeDtypeStruct((nw * n,), jnp.float32))

    scratch = [
        pltpu.VMEM((nchunks, K), jnp.int32),
        pltpu.VMEM((K,), jnp.int32),
        pltpu.VMEM((K,), jnp.int32),
        pltpu.VMEM((K,), jnp.int32),
        pltpu.VMEM((K,), jnp.int32),
        pltpu.VMEM((K, d), jnp.float32),
        pltpu.VMEM((K, d), jnp.float32),
        pltpu.VMEM_SHARED((n_pad, d), jnp.float32),
        pltpu.SemaphoreType.DMA,
        pltpu.SemaphoreType.DMA,
        pltpu.SemaphoreType.DMA,
        pltpu.SemaphoreType.DMA,
        pltpu.SemaphoreType.DMA,
        pltpu.SemaphoreType.DMA,
        pltpu.SemaphoreType.DMA,
        pltpu.SemaphoreType.DMA,
    ]
    if with_cnt:
        scratch.append(pltpu.VMEM((n,), jnp.float32))

    def body(x_hbm, src1_hbm, eidx_hbm, agg_hbm, *rest):
        if with_cnt:
            (cnt_hbm, dst2d, sib0, sib1, sib2, sib3, rows_a, rows_b, agg_sh,
             isem0, isem1, isem2, isem3, gsem_a, gsem_b,
             ssem_a, ssem_b, cnt_v) = rest
        else:
            (dst2d, sib0, sib1, sib2, sib3, rows_a, rows_b, agg_sh,
             isem0, isem1, isem2, isem3, gsem_a, gsem_b,
             ssem_a, ssem_b) = rest
        c = lax.axis_index("c")
        s = lax.axis_index("s")
        wid = s * nc + c
        ebase = wid * epw
        z16 = jnp.zeros((16,), jnp.float32)

        pltpu.sync_copy(eidx_hbm.at[wid], dst2d)

        def zrow(r, carry):
            for c8 in range(d // 16):
                rows_a[r, pl.ds(c8 * 16, 16)] = z16
            return carry
        lax.fori_loop(0, K, zrow, 0)
        row0 = s * rows_per_tile
        for j in range(nwb):
            pltpu.sync_copy(rows_a, agg_sh.at[pl.ds(row0 + j * K, K)])
        if with_cnt:
            def zcnt(i, carry):
                cnt_v[pl.ds(i * 16, 16)] = z16
                return carry
            lax.fori_loop(0, n // 16, zcnt, 0)
        plsc.subcore_barrier()

        sibs = ((sib0, isem0), (sib1, isem1), (sib2, isem2), (sib3, isem3))
        for j in range(4):
            pltpu.async_copy(src1_hbm.at[pl.ds(ebase + j * K, K)],
                             sibs[j][0], sibs[j][1])
        pltpu.make_async_copy(src1_hbm.at[pl.ds(ebase, K)],
                              sib0, isem0).wait()
        pltpu.async_copy(x_hbm.at[sib0], rows_a, gsem_a)
        ones = jnp.ones((16,), jnp.float32)
        rbufs = ((rows_a, gsem_a, ssem_a), (rows_b, gsem_b, ssem_b))

        def quad(g, carry):
            for b in range(4):
                i = 4 * g + b
                sib, isem = sibs[b]
                nsib, nisem = sibs[(b + 1) % 4]
                rows, gsem, ssem = rbufs[b % 2]
                nrows, ngsem, nssem = rbufs[(b + 1) % 2]

                @pl.when(i + 1 < nchunks)
                def _():
                    pltpu.make_async_copy(
                        src1_hbm.at[pl.ds(ebase + (i + 1) * K, K)],
                        nsib, nisem).wait()

                    @pl.when(i >= 1)
                    def _():
                        pltpu.make_async_copy(
                            nrows, agg_sh.at[dst2d.at[i]], nssem).wait()
                    pltpu.async_copy(x_hbm.at[nsib], nrows, ngsem)

                @pl.when(i < nchunks)
                def _():
                    pltpu.make_async_copy(x_hbm.at[sib], rows, gsem).wait()

                    @pl.when(i + 4 < nchunks)
                    def _():
                        pltpu.async_copy(
                            src1_hbm.at[pl.ds(ebase + (i + 4) * K, K)],
                            sib, isem)
                    pltpu.async_copy(rows, agg_sh.at[dst2d.at[i]], ssem,
                                     add=True)
                    if with_cnt:
                        for j in range(K // 16):
                            idx = dst2d[i, pl.ds(j * 16, 16)]
                            plsc.addupdate_scatter(cnt_v, [idx], ones)
            return carry
        lax.fori_loop(0, (nchunks + 3) // 4, quad, 0)
        for b in range(2):
            i_last = nchunks - 2 + b
            rows, gsem, ssem = rbufs[i_last % 2]
            pltpu.make_async_copy(
                rows, agg_sh.at[dst2d.at[i_last]], ssem).wait()
        plsc.subcore_barrier()

        for j in range(nwb):
            r = row0 + j * K
            pltpu.sync_copy(agg_sh.at[pl.ds(r, K)], rows_a)
            pltpu.sync_copy(rows_a, agg_hbm.at[c, pl.ds(r, K)])
        if with_cnt:
            pltpu.sync_copy(cnt_v, cnt_hbm.at[pl.ds(wid * n, n)])

    ot = tuple(out_type) if with_cnt else out_type[0]
    return pl.kernel(body, out_type=ot, mesh=mesh, scratch_types=scratch,
                     compiler_params=pltpu.CompilerParams(
                         needs_layout_passes=False))


@functools.lru_cache(maxsize=None)
def _make_dense(n, d, h_dim, nc, nw, relu):
    R = 1000
    grid = (n // R,)

    def body(agg_ref, cntp_ref, x_ref, wl_ref, b_ref, wr_ref, out_ref):
        cnt = jnp.sum(cntp_ref[...], axis=1)
        inv = 1.0 / jnp.maximum(cnt, 1.0)
        agg = (agg_ref[0] + agg_ref[1]) * inv[:, None]
        y = lax.dot_general(agg, wl_ref[...], (((1,), (1,)), ((), ())),
                            preferred_element_type=jnp.float32)
        y = y + b_ref[...]
        y = y + lax.dot_general(x_ref[...], wr_ref[...],
                                (((1,), (1,)), ((), ())),
                                preferred_element_type=jnp.float32)
        out_ref[...] = jnp.maximum(y, 0.0) if relu else y

    return pl.pallas_call(
        body,
        grid=grid,
        in_specs=[
            pl.BlockSpec((nc, R, d), lambda i: (0, i, 0)),
            pl.BlockSpec((R, nw), lambda i: (i, 0)),
            pl.BlockSpec((R, d), lambda i: (i, 0)),
            pl.BlockSpec((h_dim, d), lambda i: (0, 0)),
            pl.BlockSpec((1, h_dim), lambda i: (0, 0)),
            pl.BlockSpec((h_dim, d), lambda i: (0, 0)),
        ],
        out_specs=pl.BlockSpec((R, h_dim), lambda i: (i, 0)),
        out_shape=jax.ShapeDtypeStruct((n, h_dim), jnp.float32),
    )


def kernel(x, edge_index, W1_l, b1, W1_r, W2_l, b2, W2_r):
    n, d = x.shape
    e = edge_index.shape[1]
    h_dim = W1_l.shape[0]
    o_dim = W2_l.shape[0]
    info = plsc.get_sparse_core_info()
    nc, nw = info.num_cores, info.num_cores * info.num_subcores

    epw = e // nw
    K = 80
    nchunks = epw // K
    src_nodes = edge_index[0]
    dst2d = edge_index[1].reshape(nw, nchunks, K)

    aggp1, cntp = _make_sc_agg(n, d, e, True)(x, src_nodes, dst2d)
    cntp_t = cntp.reshape(nw, n).T
    h = _make_dense(n, d, h_dim, nc, nw, True)(
        aggp1, cntp_t, x, W1_l, b1.reshape(1, -1), W1_r)
    aggp2 = _make_sc_agg(n, h_dim, e, False)(h, src_nodes, dst2d)
    out = _make_dense(n, h_dim, o_dim, nc, nw, False)(
        aggp2, cntp_t, h, W2_l, b2.reshape(1, -1), W2_r)
    return (out, edge_index)
